# baseline (speedup 1.0000x reference)
import functools

import jax
import jax.numpy as jnp
from jax import lax
from jax.experimental import pallas as pl
from jax.experimental.pallas import tpu as pltpu
from jax.experimental.pallas import tpu_sc as plsc

H = 1024
FF = 512
NE = 64
NS = 1
NR = NE - NS
TOPK = 2
T = 2048

BM = 64
MAX_RBLK = 128
SH_BLK = T // BM
NBLK = MAX_RBLK + SH_BLK
RPAD = MAX_RBLK * BM
NROWS = RPAD + T


def _router_body(x_ref, wrt_ref, br_ref, wr_ref, s_out, i_out, z_out):
    i = pl.program_id(0)
    xb = x_ref[...]
    logits = jnp.dot(xb, wrt_ref[...],
                     preferred_element_type=jnp.float32)
    logits = logits + br_ref[0:1, :NE]
    col = lax.broadcasted_iota(jnp.int32, logits.shape, 1)
    valid = col < NR
    probs = jnp.where(valid, jax.nn.sigmoid(logits), -1.0)
    m1 = jnp.max(probs, axis=1, keepdims=True)
    i1 = jnp.min(jnp.where(probs == m1, col, NE), axis=1, keepdims=True)
    probs2 = jnp.where(col == i1, -1.0, probs)
    m2 = jnp.max(probs2, axis=1, keepdims=True)
    i2 = jnp.min(jnp.where(probs2 == m2, col, NE), axis=1, keepdims=True)
    den = m1 + m2
    s1 = m1 / den
    s2 = m2 / den
    c128 = lax.broadcasted_iota(jnp.int32, (xb.shape[0], 128), 1)
    zf = jnp.zeros_like(c128, dtype=jnp.float32)
    zi = jnp.zeros_like(c128)
    s_out[...] = jnp.where(c128 == 0, s1, jnp.where(c128 == 1, s2, zf))
    i_out[...] = jnp.where(c128 == 0, i1, jnp.where(c128 == 1, i2, zi))

    @pl.when(i == 0)
    def _():
        w = wr_ref[...]
        lg = jnp.log(jnp.sum(jnp.exp(w), axis=1, keepdims=True))
        rio = lax.broadcasted_iota(jnp.int32, lg.shape, 0)
        tot = jnp.sum(jnp.where(rio < NR, lg, 0.0))
        z_out[...] = jnp.full((8, 128), 0.001 * tot / NR)


def _router(x2, Wr, br):
    TB = 256
    wr_pad = jnp.zeros((NE, H), jnp.float32).at[:NR].set(Wr)
    wrt = wr_pad.T
    br_pad = jnp.zeros((8, 128), jnp.float32).at[0, :NR].set(br)
    s_out, i_out, z_out = pl.pallas_call(
        _router_body,
        grid=(T // TB,),
        in_specs=[
            pl.BlockSpec((TB, H), lambda i: (i, 0)),
            pl.BlockSpec((H, NE), lambda i: (0, 0)),
            pl.BlockSpec((8, 128), lambda i: (0, 0)),
            pl.BlockSpec((NE, H), lambda i: (0, 0)),
        ],
        out_specs=[
            pl.BlockSpec((TB, 128), lambda i: (i, 0)),
            pl.BlockSpec((TB, 128), lambda i: (i, 0)),
            pl.BlockSpec((8, 128), lambda i: (0, 0)),
        ],
        out_shape=[
            jax.ShapeDtypeStruct((T, 128), jnp.float32),
            jax.ShapeDtypeStruct((T, 128), jnp.int32),
            jax.ShapeDtypeStruct((8, 128), jnp.float32),
        ],
    )(x2, wrt, br_pad, wr_pad)
    return s_out[:, :TOPK], i_out[:, :TOPK], z_out[0, 0]


def _gemm_body(th, s_ref, xg_ref, x2_ref, wg_ref, wu_ref, wd_ref, sc_ref,
               out_ref):
    i = pl.program_id(0)
    xb = jnp.where(i >= th, x2_ref[...], xg_ref[...])
    dn = (((1,), (1,)), ((), ()))
    xbb = xb.astype(jnp.bfloat16)
    g = lax.dot_general(xbb, wg_ref[0].astype(jnp.bfloat16), dn,
                        preferred_element_type=jnp.float32)
    u = lax.dot_general(xbb, wu_ref[0].astype(jnp.bfloat16), dn,
                        preferred_element_type=jnp.float32)
    h = (g * jax.nn.sigmoid(g)) * u
    y = lax.dot_general(h.astype(jnp.bfloat16),
                        wd_ref[0].astype(jnp.bfloat16), dn,
                        preferred_element_type=jnp.float32)
    s = sc_ref[0, 0, :BM]
    out_ref[...] = y * s[:, None]


def _grouped_gemm(xg, x2, Wg_all, Wu_all, Wd_all, scores_blk, blk_expert,
                  shared_start):
    nblk = blk_expert.shape[0]
    nxg = xg.shape[0] // BM
    grid_spec = pltpu.PrefetchScalarGridSpec(
        num_scalar_prefetch=1,
        grid=(nblk,),
        in_specs=[
            pl.BlockSpec((BM, H),
                         lambda i, s: (jnp.minimum(i, nxg - 1), 0)),
            pl.BlockSpec((BM, H),
                         lambda i, s: (jnp.maximum(i - shared_start, 0), 0)),
            pl.BlockSpec((1, FF, H), lambda i, s: (s[i], 0, 0)),
            pl.BlockSpec((1, FF, H), lambda i, s: (s[i], 0, 0)),
            pl.BlockSpec((1, H, FF), lambda i, s: (s[i], 0, 0)),
            pl.BlockSpec((1, 1, 128), lambda i, s: (i, 0, 0)),
        ],
        out_specs=pl.BlockSpec((BM, H), lambda i, s: (i, 0)),
    )
    return pl.pallas_call(
        functools.partial(_gemm_body, shared_start),
        grid_spec=grid_spec,
        out_shape=jax.ShapeDtypeStruct((nblk * BM, H), jnp.float32),
    )(blk_expert, xg, x2, Wg_all, Wu_all, Wd_all, scores_blk)


NW = 32


def _sc_mesh():
    return plsc.VectorSubcoreMesh(core_axis_name="c", subcore_axis_name="s")


def _sc_gather(x2, idx_part):
    RP = idx_part.shape[0]
    rows_per_w = RP // NW
    CH = 16
    nch = rows_per_w // CH
    NB = 6

    @functools.partial(
        pl.kernel,
        out_type=jax.ShapeDtypeStruct((RP, H), jnp.float32),
        mesh=_sc_mesh(),
        scratch_types=(
            [pltpu.VMEM((rows_per_w,), jnp.int32)]
            + [pltpu.VMEM((CH, H), jnp.float32) for _ in range(NB)]
            + [pltpu.SemaphoreType.DMA for _ in range(2 * NB)]
        ),
    )
    def gk(x2_hbm, idx_hbm, out_hbm, *bufs):
        idx_full = bufs[0]
        rows = bufs[1:1 + NB]
        gsem = bufs[1 + NB:1 + 2 * NB]
        ssem = bufs[1 + 2 * NB:1 + 3 * NB]
        wid = lax.axis_index("s") * 2 + lax.axis_index("c")
        base0 = wid * rows_per_w
        pltpu.sync_copy(idx_hbm.at[pl.ds(base0, rows_per_w)], idx_full)

        def idx_sl(c):
            return idx_full.at[pl.ds(c * CH, CH)]

        def start_gather(c, k):
            pltpu.async_copy(x2_hbm.at[idx_sl(c)], rows[k], gsem[k])

        def wait_store(c, k):
            pltpu.make_async_copy(
                rows[k], out_hbm.at[pl.ds(base0 + c * CH, CH)],
                ssem[k]).wait()

        for k in range(min(NB, nch)):
            start_gather(k, k)
        for c in range(nch):
            k = c % NB
            if c >= 1 and (c - 1) + NB < nch:
                kk = (c - 1) % NB
                wait_store(c - 1, kk)
                start_gather(c - 1 + NB, kk)
            pltpu.make_async_copy(x2_hbm.at[idx_sl(c)], rows[k],
                                  gsem[k]).wait()
            pltpu.async_copy(rows[k], out_hbm.at[pl.ds(base0 + c * CH, CH)],
                             ssem[k])
        for c in range(max(nch - NB, 0), nch):
            wait_store(c, c % NB)

    return gk(x2, idx_part)


def _sc_combine(out_all, p0, p1):
    tok_per_w = T // NW
    CH = 32
    nch = tok_per_w // CH

    @functools.partial(
        pl.kernel,
        out_type=jax.ShapeDtypeStruct((T, H), jnp.float32),
        mesh=_sc_mesh(),
        scratch_types=[
            pltpu.VMEM((CH,), jnp.int32),
            pltpu.VMEM((CH,), jnp.int32),
            pltpu.VMEM((CH, H), jnp.float32),
            pltpu.VMEM((CH, H), jnp.float32),
            pltpu.VMEM((CH, H), jnp.float32),
            pltpu.SemaphoreType.DMA,
        ],
    )
    def ck(out_hbm, p0_hbm, p1_hbm, fin_hbm,
           i0_v, i1_v, acc_v, b0_v, b1_v, sem):
        wid = lax.axis_index("s") * 2 + lax.axis_index("c")
        for c in range(nch):
            base = wid * tok_per_w + c * CH
            pltpu.sync_copy(p0_hbm.at[pl.ds(base, CH)], i0_v)
            pltpu.sync_copy(p1_hbm.at[pl.ds(base, CH)], i1_v)
            cp0 = pltpu.async_copy(out_hbm.at[i0_v], b0_v, sem)
            cp1 = pltpu.async_copy(out_hbm.at[i1_v], b1_v, sem)
            pltpu.sync_copy(out_hbm.at[pl.ds(RPAD + base, CH)], acc_v)
            cp0.wait()
            cp1.wait()

            def row_body(r, _):
                def col_body(cc, _):
                    sl = pl.ds(cc * 16, 16)
                    acc_v[r, sl] = acc_v[r, sl] + b0_v[r, sl] + b1_v[r, sl]
                    return 0
                return lax.fori_loop(0, H // 16, col_body, 0)

            lax.fori_loop(0, CH, row_body, 0)
            pltpu.sync_copy(acc_v, fin_hbm.at[pl.ds(base, CH)])

    return ck(out_all, p0, p1)


def kernel(x, shared_Wg, shared_Wu, shared_Wd, Wg, Wu, Wd, Wr, br):
    Bx, Tx, C = x.shape
    x2 = x.reshape(Tx, C)

    scores2, idx2, zloss = _router(x2, Wr, br)

    e_flat = idx2.reshape(-1)
    s_flat = scores2.reshape(-1)
    order = jnp.argsort(e_flat)
    e_sorted = e_flat[order]
    tok_sorted = (order // TOPK).astype(jnp.int32)
    s_sorted = s_flat[order]
    gsz = jnp.bincount(e_sorted, length=NR)
    blocks = (gsz + BM - 1) // BM
    cum_blocks = jnp.cumsum(blocks)
    off = (cum_blocks - blocks) * BM
    gstart = jnp.cumsum(gsz) - gsz
    rank = jnp.arange(T * TOPK, dtype=jnp.int32) - gstart[e_sorted]
    pos = (off[e_sorted] + rank).astype(jnp.int32)
    idx_all = jnp.zeros((RPAD,), jnp.int32).at[pos].set(tok_sorted)
    scores_all = jnp.zeros((NROWS,), jnp.float32).at[pos].set(s_sorted)
    scores_all = scores_all.at[RPAD:].set(1.0)
    inv = jnp.zeros((T * TOPK,), jnp.int32).at[order].set(pos)
    p0 = inv[0::2]
    p1 = inv[1::2]
    blk_expert = jnp.searchsorted(
        cum_blocks, jnp.arange(MAX_RBLK, dtype=jnp.int32), side='right'
    ).astype(jnp.int32)
    blk_expert = jnp.concatenate(
        [blk_expert, jnp.full((SH_BLK,), NR, jnp.int32)])
    scores_blk = jnp.zeros((NBLK, 1, 128), jnp.float32).at[:, 0, :BM].set(
        scores_all.reshape(NBLK, BM))

    Wg_all = jnp.concatenate([Wg, shared_Wg], axis=0)
    Wu_all = jnp.concatenate([Wu, shared_Wu], axis=0)
    Wd_all = jnp.concatenate([Wd, shared_Wd], axis=0)

    HB = MAX_RBLK // 2
    xg1 = _sc_gather(x2, idx_all[:HB * BM])
    xg2 = _sc_gather(x2, idx_all[HB * BM:])
    out1 = _grouped_gemm(xg1, x2, Wg_all, Wu_all, Wd_all,
                         scores_blk[:HB], blk_expert[:HB], HB)
    out2 = _grouped_gemm(xg2, x2, Wg_all, Wu_all, Wd_all,
                         scores_blk[HB:], blk_expert[HB:], HB)
    out_all = jnp.concatenate([out1, out2], axis=0)

    final2 = _sc_combine(out_all, p0, p1)
    return final2.reshape(Bx, Tx, C), zloss


# speedup vs baseline: 1.0363x; 1.0363x over previous
import functools

import jax
import jax.numpy as jnp
from jax import lax
from jax.experimental import pallas as pl
from jax.experimental.pallas import tpu as pltpu
from jax.experimental.pallas import tpu_sc as plsc

H = 1024
FF = 512
NE = 64
NS = 1
NR = NE - NS
TOPK = 2
T = 2048

BM = 64
MAX_RBLK = 128
SH_BLK = T // BM
NBLK = MAX_RBLK + SH_BLK
RPAD = MAX_RBLK * BM
NROWS = RPAD + T


def _router_body(x_ref, wrt_ref, br_ref, wr_ref, s_out, i_out, z_out):
    i = pl.program_id(0)
    xb = x_ref[...]
    logits = jnp.dot(xb, wrt_ref[...],
                     preferred_element_type=jnp.float32)
    logits = logits + br_ref[0:1, :NE]
    col = lax.broadcasted_iota(jnp.int32, logits.shape, 1)
    valid = col < NR
    probs = jnp.where(valid, jax.nn.sigmoid(logits), -1.0)
    m1 = jnp.max(probs, axis=1, keepdims=True)
    i1 = jnp.min(jnp.where(probs == m1, col, NE), axis=1, keepdims=True)
    probs2 = jnp.where(col == i1, -1.0, probs)
    m2 = jnp.max(probs2, axis=1, keepdims=True)
    i2 = jnp.min(jnp.where(probs2 == m2, col, NE), axis=1, keepdims=True)
    den = m1 + m2
    s1 = m1 / den
    s2 = m2 / den
    c128 = lax.broadcasted_iota(jnp.int32, (xb.shape[0], 128), 1)
    zf = jnp.zeros_like(c128, dtype=jnp.float32)
    zi = jnp.zeros_like(c128)
    s_out[...] = jnp.where(c128 == 0, s1, jnp.where(c128 == 1, s2, zf))
    i_out[...] = jnp.where(c128 == 0, i1, jnp.where(c128 == 1, i2, zi))

    @pl.when(i == 0)
    def _():
        w = wr_ref[...]
        lg = jnp.log(jnp.sum(jnp.exp(w), axis=1, keepdims=True))
        rio = lax.broadcasted_iota(jnp.int32, lg.shape, 0)
        tot = jnp.sum(jnp.where(rio < NR, lg, 0.0))
        z_out[...] = jnp.full((8, 128), 0.001 * tot / NR)


def _router(x2, Wr, br):
    TB = 256
    wr_pad = jnp.zeros((NE, H), jnp.float32).at[:NR].set(Wr)
    wrt = wr_pad.T
    br_pad = jnp.zeros((8, 128), jnp.float32).at[0, :NR].set(br)
    s_out, i_out, z_out = pl.pallas_call(
        _router_body,
        grid=(T // TB,),
        in_specs=[
            pl.BlockSpec((TB, H), lambda i: (i, 0)),
            pl.BlockSpec((H, NE), lambda i: (0, 0)),
            pl.BlockSpec((8, 128), lambda i: (0, 0)),
            pl.BlockSpec((NE, H), lambda i: (0, 0)),
        ],
        out_specs=[
            pl.BlockSpec((TB, 128), lambda i: (i, 0)),
            pl.BlockSpec((TB, 128), lambda i: (i, 0)),
            pl.BlockSpec((8, 128), lambda i: (0, 0)),
        ],
        out_shape=[
            jax.ShapeDtypeStruct((T, 128), jnp.float32),
            jax.ShapeDtypeStruct((T, 128), jnp.int32),
            jax.ShapeDtypeStruct((8, 128), jnp.float32),
        ],
    )(x2, wrt, br_pad, wr_pad)
    return s_out[:, :TOPK], i_out[:, :TOPK], z_out[0, 0]


def _gemm_body(th, s_ref, xg_ref, x2_ref, wg_ref, wu_ref, wd_ref, sc_ref,
               out_ref):
    i = pl.program_id(0)
    xb = jnp.where(i >= th, x2_ref[...], xg_ref[...])
    dn = (((1,), (1,)), ((), ()))
    xbb = xb.astype(jnp.bfloat16)
    g = lax.dot_general(xbb, wg_ref[0].astype(jnp.bfloat16), dn,
                        preferred_element_type=jnp.float32)
    u = lax.dot_general(xbb, wu_ref[0].astype(jnp.bfloat16), dn,
                        preferred_element_type=jnp.float32)
    h = (g * jax.nn.sigmoid(g)) * u
    y = lax.dot_general(h.astype(jnp.bfloat16),
                        wd_ref[0].astype(jnp.bfloat16), dn,
                        preferred_element_type=jnp.float32)
    s = sc_ref[0, 0, :BM]
    out_ref[...] = y * s[:, None]


def _grouped_gemm(xg, x2, Wg_all, Wu_all, Wd_all, scores_blk, blk_expert,
                  shared_start):
    nblk = blk_expert.shape[0]
    nxg = xg.shape[0] // BM
    grid_spec = pltpu.PrefetchScalarGridSpec(
        num_scalar_prefetch=1,
        grid=(nblk,),
        in_specs=[
            pl.BlockSpec((BM, H),
                         lambda i, s: (jnp.minimum(i, nxg - 1), 0)),
            pl.BlockSpec((BM, H),
                         lambda i, s: (jnp.maximum(i - shared_start, 0), 0)),
            pl.BlockSpec((1, FF, H), lambda i, s: (s[i], 0, 0)),
            pl.BlockSpec((1, FF, H), lambda i, s: (s[i], 0, 0)),
            pl.BlockSpec((1, H, FF), lambda i, s: (s[i], 0, 0)),
            pl.BlockSpec((1, 1, 128), lambda i, s: (i, 0, 0)),
        ],
        out_specs=pl.BlockSpec((BM, H), lambda i, s: (i, 0)),
    )
    return pl.pallas_call(
        functools.partial(_gemm_body, shared_start),
        grid_spec=grid_spec,
        out_shape=jax.ShapeDtypeStruct((nblk * BM, H), jnp.float32),
    )(blk_expert, xg, x2, Wg_all, Wu_all, Wd_all, scores_blk)


NW = 32


def _sc_mesh():
    return plsc.VectorSubcoreMesh(core_axis_name="c", subcore_axis_name="s")


def _sc_gather(x2, idx_part):
    RP = idx_part.shape[0]
    rows_per_w = RP // NW
    CH = 32
    nch = rows_per_w // CH
    NB = 3

    @functools.partial(
        pl.kernel,
        out_type=jax.ShapeDtypeStruct((RP, H), jnp.float32),
        mesh=_sc_mesh(),
        scratch_types=(
            [pltpu.VMEM((rows_per_w,), jnp.int32)]
            + [pltpu.VMEM((CH, H), jnp.float32) for _ in range(NB)]
            + [pltpu.SemaphoreType.DMA for _ in range(2 * NB)]
        ),
    )
    def gk(x2_hbm, idx_hbm, out_hbm, *bufs):
        idx_full = bufs[0]
        rows = bufs[1:1 + NB]
        gsem = bufs[1 + NB:1 + 2 * NB]
        ssem = bufs[1 + 2 * NB:1 + 3 * NB]
        wid = lax.axis_index("s") * 2 + lax.axis_index("c")
        base0 = wid * rows_per_w
        pltpu.sync_copy(idx_hbm.at[pl.ds(base0, rows_per_w)], idx_full)

        def idx_sl(c):
            return idx_full.at[pl.ds(c * CH, CH)]

        def start_gather(c, k):
            pltpu.async_copy(x2_hbm.at[idx_sl(c)], rows[k], gsem[k])

        def wait_store(c, k):
            pltpu.make_async_copy(
                rows[k], out_hbm.at[pl.ds(base0 + c * CH, CH)],
                ssem[k]).wait()

        for k in range(min(NB, nch)):
            start_gather(k, k)
        for c in range(nch):
            k = c % NB
            if c >= 1 and (c - 1) + NB < nch:
                kk = (c - 1) % NB
                wait_store(c - 1, kk)
                start_gather(c - 1 + NB, kk)
            pltpu.make_async_copy(x2_hbm.at[idx_sl(c)], rows[k],
                                  gsem[k]).wait()
            pltpu.async_copy(rows[k], out_hbm.at[pl.ds(base0 + c * CH, CH)],
                             ssem[k])
        for c in range(max(nch - NB, 0), nch):
            wait_store(c, c % NB)

    return gk(x2, idx_part)


def _sc_combine(out_all, p0, p1):
    tok_per_w = T // NW
    CH = 32
    nch = tok_per_w // CH

    @functools.partial(
        pl.kernel,
        out_type=jax.ShapeDtypeStruct((T, H), jnp.float32),
        mesh=_sc_mesh(),
        scratch_types=[
            pltpu.VMEM((CH,), jnp.int32),
            pltpu.VMEM((CH,), jnp.int32),
            pltpu.VMEM((CH, H), jnp.float32),
            pltpu.VMEM((CH, H), jnp.float32),
            pltpu.VMEM((CH, H), jnp.float32),
            pltpu.SemaphoreType.DMA,
        ],
    )
    def ck(out_hbm, p0_hbm, p1_hbm, fin_hbm,
           i0_v, i1_v, acc_v, b0_v, b1_v, sem):
        wid = lax.axis_index("s") * 2 + lax.axis_index("c")
        for c in range(nch):
            base = wid * tok_per_w + c * CH
            pltpu.sync_copy(p0_hbm.at[pl.ds(base, CH)], i0_v)
            pltpu.sync_copy(p1_hbm.at[pl.ds(base, CH)], i1_v)
            cp0 = pltpu.async_copy(out_hbm.at[i0_v], b0_v, sem)
            cp1 = pltpu.async_copy(out_hbm.at[i1_v], b1_v, sem)
            pltpu.sync_copy(out_hbm.at[pl.ds(RPAD + base, CH)], acc_v)
            cp0.wait()
            cp1.wait()

            def row_body(r, _):
                def col_body(cc, _):
                    sl = pl.ds(cc * 16, 16)
                    acc_v[r, sl] = acc_v[r, sl] + b0_v[r, sl] + b1_v[r, sl]
                    return 0
                return lax.fori_loop(0, H // 16, col_body, 0)

            lax.fori_loop(0, CH, row_body, 0)
            pltpu.sync_copy(acc_v, fin_hbm.at[pl.ds(base, CH)])

    return ck(out_all, p0, p1)


def kernel(x, shared_Wg, shared_Wu, shared_Wd, Wg, Wu, Wd, Wr, br):
    Bx, Tx, C = x.shape
    x2 = x.reshape(Tx, C)

    scores2, idx2, zloss = _router(x2, Wr, br)

    e_flat = idx2.reshape(-1)
    s_flat = scores2.reshape(-1)
    order = jnp.argsort(e_flat)
    e_sorted = e_flat[order]
    tok_sorted = (order // TOPK).astype(jnp.int32)
    s_sorted = s_flat[order]
    gsz = jnp.bincount(e_sorted, length=NR)
    blocks = (gsz + BM - 1) // BM
    cum_blocks = jnp.cumsum(blocks)
    off = (cum_blocks - blocks) * BM
    gstart = jnp.cumsum(gsz) - gsz
    rank = jnp.arange(T * TOPK, dtype=jnp.int32) - gstart[e_sorted]
    pos = (off[e_sorted] + rank).astype(jnp.int32)
    idx_all = jnp.zeros((RPAD,), jnp.int32).at[pos].set(tok_sorted)
    scores_all = jnp.zeros((NROWS,), jnp.float32).at[pos].set(s_sorted)
    scores_all = scores_all.at[RPAD:].set(1.0)
    inv = jnp.zeros((T * TOPK,), jnp.int32).at[order].set(pos)
    p0 = inv[0::2]
    p1 = inv[1::2]
    blk_expert = jnp.searchsorted(
        cum_blocks, jnp.arange(MAX_RBLK, dtype=jnp.int32), side='right'
    ).astype(jnp.int32)
    blk_expert = jnp.concatenate(
        [blk_expert, jnp.full((SH_BLK,), NR, jnp.int32)])
    scores_blk = jnp.zeros((NBLK, 1, 128), jnp.float32).at[:, 0, :BM].set(
        scores_all.reshape(NBLK, BM))

    Wg_all = jnp.concatenate([Wg, shared_Wg], axis=0)
    Wu_all = jnp.concatenate([Wu, shared_Wu], axis=0)
    Wd_all = jnp.concatenate([Wd, shared_Wd], axis=0)

    xg = _sc_gather(x2, idx_all)

    out_all = _grouped_gemm(xg, x2, Wg_all, Wu_all, Wd_all,
                            scores_blk, blk_expert, MAX_RBLK)

    final2 = _sc_combine(out_all, p0, p1)
    return final2.reshape(Bx, Tx, C), zloss
